# Initial kernel scaffold; baseline (speedup 1.0000x reference)
import jax
import jax.numpy as jnp
from jax.experimental import pallas as pl


def kernel(x1, edge_index1, e1, u1, batch1, x2, edge_index2, e2, u2, batch2, params):
    raise NotImplementedError("write your pallas kernel here")


# speedup vs baseline: 1.8552x; 1.8552x over previous
import functools
import jax
import jax.numpy as jnp
from jax.experimental import pallas as pl
from jax.experimental.pallas import tpu as pltpu

H = 32
HID = 64
EB = 2000
NB_ = 2000


def _relu(v):
    return jnp.maximum(v, 0.0)


def _edge_mlp_call(n_rows, mode):
    grid = (n_rows // EB,)

    def body(*refs):
        if mode == 'enc':
            (e_ref, d_ref, c0_ref, A_ref, W2_ref, b2_ref, W3_ref, b3_ref,
             Be_ref, We1_ref, be1_ref, We2_ref, be2_ref, We3_ref, be3_ref,
             out_ref, s_ref) = refs
        elif mode == 'mid':
            (e_ref, eh_ref, d_ref, c0_ref, A_ref, B_ref, W2_ref, b2_ref,
             W3_ref, b3_ref, out_ref, s_ref) = refs
        else:
            (eh_ref, d_ref, c0_ref, B_ref, W2_ref, b2_ref, W3_ref, b3_ref,
             out_ref, s_ref, sw_ref) = refs

        i = pl.program_id(0)
        f32 = jnp.float32
        if mode == 'enc':
            e = e_ref[...]
            z = _relu(jnp.dot(e, We1_ref[...], preferred_element_type=f32) + be1_ref[...])
            z = _relu(jnp.dot(z, We2_ref[...], preferred_element_type=f32) + be2_ref[...])
            eh = jnp.dot(z, We3_ref[...], preferred_element_type=f32) + be3_ref[...]
            h1 = (jnp.dot(e, A_ref[...], preferred_element_type=f32)
                  + jnp.dot(eh, Be_ref[...], preferred_element_type=f32)
                  + d_ref[...] + c0_ref[...])
        elif mode == 'mid':
            eh = eh_ref[...]
            h1 = (jnp.dot(e_ref[...], A_ref[...], preferred_element_type=f32)
                  + jnp.dot(eh, B_ref[...], preferred_element_type=f32)
                  + d_ref[...] + c0_ref[...])
        else:
            eh = eh_ref[...]
            h1 = (jnp.dot(eh, B_ref[...], preferred_element_type=f32)
                  + d_ref[...] + c0_ref[...])
        z = _relu(h1)
        z = _relu(jnp.dot(z, W2_ref[...], preferred_element_type=f32) + b2_ref[...])
        out = jnp.dot(z, W3_ref[...], preferred_element_type=f32) + b3_ref[...]
        out_ref[...] = out

        @pl.when(i == 0)
        def _():
            s_ref[...] = jnp.zeros_like(s_ref)
            if mode == 'att':
                sw_ref[...] = jnp.zeros_like(sw_ref)

        s_ref[...] += jnp.sum(out, axis=0, keepdims=True)
        if mode == 'att':
            sw_ref[...] += jnp.sum(out * eh, axis=0, keepdims=True)

    row = lambda w: pl.BlockSpec((EB, w), lambda i: (i, 0))
    full = lambda a, b: pl.BlockSpec((a, b), lambda i: (0, 0))

    if mode == 'enc':
        in_specs = [row(16), row(64), full(1, 64), full(16, 64),
                    full(64, 64), full(1, 64), full(64, 32), full(1, 32),
                    full(32, 64), full(16, 64), full(1, 64), full(64, 64),
                    full(1, 64), full(64, 32), full(1, 32)]
    elif mode == 'mid':
        in_specs = [row(16), row(32), row(64), full(1, 64), full(16, 64),
                    full(32, 64), full(64, 64), full(1, 64), full(64, 32),
                    full(1, 32)]
    else:
        in_specs = [row(32), row(64), full(1, 64), full(32, 64),
                    full(64, 64), full(1, 64), full(64, 32), full(1, 32)]

    out_specs = [row(32), full(1, 32)]
    out_shape = [jax.ShapeDtypeStruct((n_rows, 32), jnp.float32),
                 jax.ShapeDtypeStruct((1, 32), jnp.float32)]
    if mode == 'att':
        out_specs.append(full(1, 32))
        out_shape.append(jax.ShapeDtypeStruct((1, 32), jnp.float32))

    return pl.pallas_call(body, grid=grid, in_specs=in_specs,
                          out_specs=out_specs, out_shape=out_shape)


def _node_enc_call(n_rows):
    grid = (n_rows // NB_,)

    def body(x_ref, W1_ref, b1_ref, W2_ref, b2_ref, W3_ref, b3_ref,
             Wxx_ref, Nx_ref, Wxh_ref,
             xh_ref, t_ref, xn_ref, yx_ref):
        f32 = jnp.float32
        x = x_ref[...]
        z = _relu(jnp.dot(x, W1_ref[...], preferred_element_type=f32) + b1_ref[...])
        z = _relu(jnp.dot(z, W2_ref[...], preferred_element_type=f32) + b2_ref[...])
        xh = jnp.dot(z, W3_ref[...], preferred_element_type=f32) + b3_ref[...]
        yx = jnp.dot(x, Wxx_ref[...], preferred_element_type=f32)
        xh_ref[...] = xh
        yx_ref[...] = yx
        t_ref[...] = yx + jnp.dot(xh, Wxh_ref[...], preferred_element_type=f32)
        xn_ref[...] = jnp.dot(x, Nx_ref[...], preferred_element_type=f32)

    row = lambda w: pl.BlockSpec((NB_, w), lambda i: (i, 0))
    full = lambda a, b: pl.BlockSpec((a, b), lambda i: (0, 0))
    in_specs = [row(128), full(128, 64), full(1, 64), full(64, 64), full(1, 64),
                full(64, 32), full(1, 32), full(128, 64), full(128, 64),
                full(32, 64)]
    out_specs = [row(32), row(64), row(64), row(64)]
    out_shape = [jax.ShapeDtypeStruct((n_rows, 32), jnp.float32),
                 jax.ShapeDtypeStruct((n_rows, 64), jnp.float32),
                 jax.ShapeDtypeStruct((n_rows, 64), jnp.float32),
                 jax.ShapeDtypeStruct((n_rows, 64), jnp.float32)]
    return pl.pallas_call(body, grid=grid, in_specs=in_specs,
                          out_specs=out_specs, out_shape=out_shape)


def _node_mlp_call(n_rows, mode):
    grid = (n_rows // NB_,)

    def body(*refs):
        if mode == 'rec':
            (xn_ref, xh_ref, s0_ref, s1_ref, cnt_ref, yx_ref, c0_ref,
             Nxh_ref, Na_ref, W2_ref, b2_ref, W3_ref, b3_ref, Wt_ref,
             out_ref, t_ref, s_ref) = refs
        else:
            (xh_ref, s0_ref, s1_ref, cnt_ref, c0_ref,
             Nxh_ref, Na_ref, W2_ref, b2_ref, W3_ref, b3_ref,
             out_ref, s_ref, sw_ref) = refs
        i = pl.program_id(0)
        f32 = jnp.float32
        xh = xh_ref[...]
        inv = 1.0 / jnp.maximum(cnt_ref[...], 1.0)
        agg = (s0_ref[...] + s1_ref[...]) * inv
        h1 = (jnp.dot(xh, Nxh_ref[...], preferred_element_type=f32)
              + jnp.dot(agg, Na_ref[...], preferred_element_type=f32)
              + c0_ref[...])
        if mode == 'rec':
            h1 = h1 + xn_ref[...]
        z = _relu(h1)
        z = _relu(jnp.dot(z, W2_ref[...], preferred_element_type=f32) + b2_ref[...])
        out = jnp.dot(z, W3_ref[...], preferred_element_type=f32) + b3_ref[...]
        out_ref[...] = out
        if mode == 'rec':
            t_ref[...] = yx_ref[...] + jnp.dot(out, Wt_ref[...],
                                               preferred_element_type=f32)

        @pl.when(i == 0)
        def _():
            s_ref[...] = jnp.zeros_like(s_ref)
            if mode == 'att':
                sw_ref[...] = jnp.zeros_like(sw_ref)

        s_ref[...] += jnp.sum(out, axis=0, keepdims=True)
        if mode == 'att':
            sw_ref[...] += jnp.sum(out * xh, axis=0, keepdims=True)

    row = lambda w: pl.BlockSpec((NB_, w), lambda i: (i, 0))
    full = lambda a, b: pl.BlockSpec((a, b), lambda i: (0, 0))
    if mode == 'rec':
        in_specs = [row(64), row(32), row(32), row(32), row(1), row(64),
                    full(1, 64), full(32, 64), full(32, 64), full(64, 64),
                    full(1, 64), full(64, 32), full(1, 32), full(32, 64)]
        out_specs = [row(32), row(64), full(1, 32)]
        out_shape = [jax.ShapeDtypeStruct((n_rows, 32), jnp.float32),
                     jax.ShapeDtypeStruct((n_rows, 64), jnp.float32),
                     jax.ShapeDtypeStruct((1, 32), jnp.float32)]
    else:
        in_specs = [row(32), row(32), row(32), row(1),
                    full(1, 64), full(32, 64), full(32, 64), full(64, 64),
                    full(1, 64), full(64, 32), full(1, 32)]
        out_specs = [row(32), full(1, 32), full(1, 32)]
        out_shape = [jax.ShapeDtypeStruct((n_rows, 32), jnp.float32),
                     jax.ShapeDtypeStruct((1, 32), jnp.float32),
                     jax.ShapeDtypeStruct((1, 32), jnp.float32)]
    return pl.pallas_call(body, grid=grid, in_specs=in_specs,
                          out_specs=out_specs, out_shape=out_shape)


def _gather_diff(t, src, dst):
    return jnp.take(t, dst, axis=0) - jnp.take(t, src, axis=0)


def _segment_sum(vals, dst, n):
    return jnp.zeros((n, vals.shape[1]), jnp.float32).at[dst].add(vals)


def _mlp_rows(params, v):
    for W, b in params[:-1]:
        v = _relu(jnp.dot(v, W) + b)
    W, b = params[-1]
    return jnp.dot(v, W) + b


def _graph_embed(params, x, edge_index, e, u):
    n = x.shape[0]
    n_edges = e.shape[0]
    src, dst = edge_index[0], edge_index[1]

    W1e, b1e = params['rec_edge'][0]
    A = W1e[0:16]; B = W1e[16:48]; Wxx = W1e[48:176]; Wxh = W1e[176:208]
    Wu = W1e[208:224]; Wuh = W1e[224:256]
    W1n, b1n = params['rec_node'][0]
    Nx = W1n[0:128]; Nxh = W1n[128:160]; Na = W1n[160:192]
    Nu = W1n[192:208]; Nuh = W1n[208:240]
    W1ae, b1ae = params['att_edge'][0]
    AB = W1ae[0:32]; AC = W1ae[32:64]; AU = W1ae[64:96]
    W1an, b1an = params['att_node'][0]
    ANx = W1an[0:32]; ANa = W1an[32:64]; ANu = W1an[64:96]

    (We2, be2), (We3, be3) = params['rec_edge'][1], params['rec_edge'][2]
    (Wn2, bn2), (Wn3, bn3) = params['rec_node'][1], params['rec_node'][2]
    (Wae2, bae2), (Wae3, bae3) = params['att_edge'][1], params['att_edge'][2]
    (Wan2, ban2), (Wan3, ban3) = params['att_node'][1], params['att_node'][2]
    enc_e = params['enc_edge']
    enc_n = params['enc_node']

    r2 = lambda v: v.reshape(1, -1)
    u_h = _mlp_rows(params['enc_glob'], u)

    xh0, t, xn, yx = _node_enc_call(n)(
        x, enc_n[0][0], r2(enc_n[0][1]), enc_n[1][0], r2(enc_n[1][1]),
        enc_n[2][0], r2(enc_n[2][1]), Wxx, Nx, Wxh)

    cnt = _segment_sum(jnp.ones((n_edges, 1), jnp.float32), dst, n)

    e_h = None
    for r in range(3):
        c0e = b1e.reshape(1, -1) + u @ Wu + u_h @ Wuh
        d = _gather_diff(t, src, dst)
        if r == 0:
            e_h, esum = _edge_mlp_call(n_edges, 'enc')(
                e, d, c0e, A, We2, r2(be2), We3, r2(be3),
                B, enc_e[0][0], r2(enc_e[0][1]), enc_e[1][0], r2(enc_e[1][1]),
                enc_e[2][0], r2(enc_e[2][1]))
        else:
            e_h, esum = _edge_mlp_call(n_edges, 'mid')(
                e, e_h, d, c0e, A, B, We2, r2(be2), We3, r2(be3))
        ssum = _segment_sum(e_h, dst, n)
        c0n = b1n.reshape(1, -1) + u @ Nu + u_h @ Nuh
        Wt = Wxh if r < 2 else AC
        x_h, t, xsum = _node_mlp_call(n, 'rec')(
            xn, (xh0 if r == 0 else x_h), ssum,
            jnp.zeros_like(ssum), cnt, (yx if r < 2 else jnp.zeros_like(yx)),
            c0n, Nxh, Na, Wn2, r2(bn2), Wn3, r2(bn3), Wt)
        gin = jnp.concatenate([esum / n_edges, xsum / n, u, u_h], axis=1)
        u_h = _mlp_rows(params['rec_glob'], gin)

    c0ae = b1ae.reshape(1, -1) + u_h @ AU
    d = _gather_diff(t, src, dst)
    e_a, easum, ewsum = _edge_mlp_call(n_edges, 'att')(
        e_h, d, c0ae, AB, Wae2, r2(bae2), Wae3, r2(bae3))
    ssa = _segment_sum(e_a, dst, n)
    c0an = b1an.reshape(1, -1) + u_h @ ANu
    x_a, xasum, xwsum = _node_mlp_call(n, 'att')(
        x_h, ssa, jnp.zeros_like(ssa), cnt, c0an, ANx, ANa,
        Wan2, r2(ban2), Wan3, r2(ban3))

    u_a = _mlp_rows(params['att_glob'],
                    jnp.concatenate([easum / n_edges, xasum / n, u_h], axis=1))
    u_w = u_a * u_h
    gin = jnp.concatenate([ewsum / n_edges, xwsum / n, u_w], axis=1)
    return _mlp_rows(params['agg'], gin)


def kernel(x1, edge_index1, e1, u1, batch1, x2, edge_index2, e2, u2, batch2,
           params):
    ue1 = _graph_embed(params, x1, edge_index1, e1, u1)
    ue2 = _graph_embed(params, x2, edge_index2, e2, u2)
    return _mlp_rows(params['final'], ue1 - ue2)
